# baseline (speedup 1.0000x reference)
import functools

import jax
import jax.numpy as jnp
from jax import lax
from jax.experimental import pallas as pl
from jax.experimental.pallas import tpu as pltpu
from jax.experimental.pallas import tpu_sc as plsc

_CHUNK = 32


def _sc_broadcast_add(inputs, table):
    batch, seq, dim = inputs.shape
    info = plsc.get_sparse_core_info()
    nc, ns, nl = info.num_cores, info.num_subcores, info.num_lanes
    nw = nc * ns
    s_per_w = seq // nw
    n_chunks = s_per_w // _CHUNK
    mesh = plsc.VectorSubcoreMesh(core_axis_name="c", subcore_axis_name="s")

    @functools.partial(
        pl.kernel,
        mesh=mesh,
        out_type=jax.ShapeDtypeStruct((batch, seq, dim), jnp.float32),
        scratch_types=[
            pltpu.VMEM((_CHUNK, dim), jnp.float32),
            pltpu.VMEM((_CHUNK, dim), jnp.float32),
        ],
    )
    def k(in_hbm, tab_hbm, out_hbm, tab_v, buf_v):
        wid = lax.axis_index("s") * nc + lax.axis_index("c")
        s0 = wid * s_per_w
        for c in range(n_chunks):
            base = s0 + c * _CHUNK
            pltpu.sync_copy(tab_hbm.at[pl.ds(base, _CHUNK)], tab_v)
            for b in range(batch):
                pltpu.sync_copy(in_hbm.at[b, pl.ds(base, _CHUNK)], buf_v)

                def row(r, carry):
                    for u in range(dim // nl):
                        sl = pl.ds(u * nl, nl)
                        buf_v[r, sl] = buf_v[r, sl] + tab_v[r, sl]
                    return carry

                lax.fori_loop(0, _CHUNK, row, 0)
                pltpu.sync_copy(buf_v, out_hbm.at[b, pl.ds(base, _CHUNK)])

    return k(inputs, table)


def kernel(inputs, position_table):
    return _sc_broadcast_add(inputs, position_table)


# speedup vs baseline: 1.2369x; 1.2369x over previous
import functools

import jax
import jax.numpy as jnp
from jax import lax
from jax.experimental import pallas as pl
from jax.experimental.pallas import tpu as pltpu
from jax.experimental.pallas import tpu_sc as plsc

_CHUNK = 32


def _sc_broadcast_add(inputs, table):
    batch, seq, dim = inputs.shape
    info = plsc.get_sparse_core_info()
    nc, ns, nl = info.num_cores, info.num_subcores, info.num_lanes
    nw = nc * ns
    s_per_w = seq // nw
    n_chunks = s_per_w // _CHUNK
    n_iter = n_chunks * batch
    mesh = plsc.VectorSubcoreMesh(core_axis_name="c", subcore_axis_name="s")

    @functools.partial(
        pl.kernel,
        mesh=mesh,
        out_type=jax.ShapeDtypeStruct((batch, seq, dim), jnp.float32),
        scratch_types=[
            pltpu.VMEM((_CHUNK, dim), jnp.float32),
            pltpu.VMEM((_CHUNK, dim), jnp.float32),
            pltpu.VMEM((_CHUNK, dim), jnp.float32),
            pltpu.SemaphoreType.DMA,
            pltpu.SemaphoreType.DMA,
            pltpu.SemaphoreType.DMA,
            pltpu.SemaphoreType.DMA,
        ],
    )
    def k(in_hbm, tab_hbm, out_hbm, tab_v, in0, in1, ld0, ld1, st0, st1):
        wid = lax.axis_index("s") * nc + lax.axis_index("c")
        s0 = wid * s_per_w
        bufs = (in0, in1)
        lsems = (ld0, ld1)
        ssems = (st0, st1)

        def src(i):
            c, b = divmod(i, batch)
            return in_hbm.at[b, pl.ds(s0 + c * _CHUNK, _CHUNK)]

        def dst(i):
            c, b = divmod(i, batch)
            return out_hbm.at[b, pl.ds(s0 + c * _CHUNK, _CHUNK)]

        pltpu.make_async_copy(src(0), bufs[0], lsems[0]).start()
        for i in range(n_iter):
            c, b = divmod(i, batch)
            p = i % 2
            if i + 1 < n_iter:
                q = (i + 1) % 2
                if i >= 1:
                    pltpu.make_async_copy(bufs[q], dst(i - 1), ssems[q]).wait()
                pltpu.make_async_copy(src(i + 1), bufs[q], lsems[q]).start()
            if b == 0:
                pltpu.sync_copy(tab_hbm.at[pl.ds(s0 + c * _CHUNK, _CHUNK)], tab_v)
            pltpu.make_async_copy(src(i), bufs[p], lsems[p]).wait()
            buf = bufs[p]

            def row(r, carry, buf=buf):
                for u in range(dim // nl):
                    sl = pl.ds(u * nl, nl)
                    buf[r, sl] = buf[r, sl] + tab_v[r, sl]
                return carry

            lax.fori_loop(0, _CHUNK, row, 0)
            pltpu.make_async_copy(buf, dst(i), ssems[p]).start()
        for i in (n_iter - 2, n_iter - 1):
            pltpu.make_async_copy(bufs[i % 2], dst(i), ssems[i % 2]).wait()

    return k(inputs, table)


def kernel(inputs, position_table):
    return _sc_broadcast_add(inputs, position_table)


# speedup vs baseline: 2.5878x; 2.0922x over previous
import jax
import jax.numpy as jnp
from jax.experimental import pallas as pl

_BS = 512


def _tc_broadcast_add(inputs, table):
    batch, seq, dim = inputs.shape

    def body(in_ref, tab_ref, out_ref):
        out_ref[...] = in_ref[...] + tab_ref[...]

    return pl.pallas_call(
        body,
        grid=(seq // _BS, batch),
        in_specs=[
            pl.BlockSpec((1, _BS, dim), lambda i, b: (b, i, 0)),
            pl.BlockSpec((_BS, dim), lambda i, b: (i, 0)),
        ],
        out_specs=pl.BlockSpec((1, _BS, dim), lambda i, b: (b, i, 0)),
        out_shape=jax.ShapeDtypeStruct((batch, seq, dim), jnp.float32),
    )(inputs, table)


def kernel(inputs, position_table):
    return _tc_broadcast_add(inputs, position_table)
